# baseline (speedup 1.0000x reference)
import functools

import jax
import jax.numpy as jnp
from jax import lax
from jax.experimental import pallas as pl
from jax.experimental.pallas import tpu as pltpu
from jax.experimental.pallas import tpu_sc as plsc

MARGIN = 1.0
EPS = 1e-6

D = 512
B = 16384
L = 16
NW = 32
BPW = B // NW
C = 16
NBUF = 2
NCHUNK = BPW // C
DV = D // L
PROWS = BPW // 8

_mesh = plsc.VectorSubcoreMesh(core_axis_name="c", subcore_axis_name="s")


@functools.partial(
    pl.kernel,
    mesh=_mesh,
    compiler_params=pltpu.CompilerParams(use_tc_tiling_on_sc=False),
    out_type=[
        jax.ShapeDtypeStruct((NW * PROWS, 128), jnp.float32),
        jax.ShapeDtypeStruct((NW * PROWS, 128), jnp.float32),
    ],
    scratch_types=[
        pltpu.VMEM((BPW,), jnp.int32),
        pltpu.VMEM((BPW,), jnp.int32),
        pltpu.VMEM((BPW,), jnp.int32),
        *([pltpu.VMEM((C, D), jnp.float32)] * 6),
        pltpu.VMEM((PROWS, 128), jnp.float32),
        pltpu.VMEM((PROWS, 128), jnp.float32),
        *([pltpu.SemaphoreType.DMA] * 2),
    ],
)
def _sc_ssd(table_hbm, ia_hbm, ip_hbm, in_hbm, outp_hbm, outn_hbm,
            ia_v, ip_v, in_v,
            ra0, rp0, rn0, ra1, rp1, rn1,
            ssdp_v, ssdn_v, sem0, sem1):
    wid = lax.axis_index("s") * 2 + lax.axis_index("c")
    pltpu.sync_copy(ia_hbm.at[pl.ds(wid * BPW, BPW)], ia_v)
    pltpu.sync_copy(ip_hbm.at[pl.ds(wid * BPW, BPW)], ip_v)
    pltpu.sync_copy(in_hbm.at[pl.ds(wid * BPW, BPW)], in_v)

    slots = ((ra0, rp0, rn0, sem0), (ra1, rp1, rn1, sem1))

    def issue(g, slot):
        ra, rp, rn, sem = slot
        base = g * C
        pltpu.async_copy(table_hbm.at[ia_v.at[pl.ds(base, C)]], ra, sem)
        pltpu.async_copy(table_hbm.at[ip_v.at[pl.ds(base, C)]], rp, sem)
        pltpu.async_copy(table_hbm.at[in_v.at[pl.ds(base, C)]], rn, sem)

    def wait_slot(slot):
        ra, rp, rn, sem = slot
        dummy = table_hbm.at[pl.ds(0, C)]
        pltpu.make_async_copy(dummy, ra, sem).wait()
        pltpu.make_async_copy(dummy, rp, sem).wait()
        pltpu.make_async_copy(dummy, rn, sem).wait()

    def compute(g, slot):
        ra, rp, rn, _ = slot
        base = g * C

        def trip_body(i, carry2):
            accp = jnp.zeros((L,), jnp.float32)
            accn = jnp.zeros((L,), jnp.float32)
            for j in range(DV):
                a = ra[i, pl.ds(j * L, L)]
                p = rp[i, pl.ds(j * L, L)]
                n = rn[i, pl.ds(j * L, L)]
                t = a + EPS
                dp = t - p
                dn = t - n
                accp = accp + dp * dp
                accn = accn + dn * dn
            t2 = base + i
            row = t2 // 8
            col = (t2 % 8) * L
            ssdp_v[row, pl.ds(col, L)] = accp
            ssdn_v[row, pl.ds(col, L)] = accn
            return carry2

        lax.fori_loop(0, C, trip_body, 0)

    for s in range(NBUF):
        issue(s, slots[s])

    def body4(gg, carry):
        g0 = NBUF * gg
        for s in range(NBUF):
            g = g0 + s
            wait_slot(slots[s])
            compute(g, slots[s])

            @pl.when(g + NBUF < NCHUNK)
            def _():
                issue(g + NBUF, slots[s])

        return carry

    lax.fori_loop(0, NCHUNK // NBUF, body4, 0)

    pltpu.sync_copy(ssdp_v, outp_hbm.at[pl.ds(wid * PROWS, PROWS)])
    pltpu.sync_copy(ssdn_v, outn_hbm.at[pl.ds(wid * PROWS, PROWS)])


def _tc_finish_body(pp_ref, pn_ref, out_ref):
    lanes = lax.broadcasted_iota(jnp.int32, (128, 8), 0) // L
    groups = lax.broadcasted_iota(jnp.int32, (128, 8), 1)
    g_mat = (lanes == groups).astype(jnp.float32)
    sp = jnp.dot(pp_ref[...], g_mat, preferred_element_type=jnp.float32)
    sn = jnp.dot(pn_ref[...], g_mat, preferred_element_type=jnp.float32)
    dp = sp * lax.rsqrt(jnp.maximum(sp, 1e-30))
    dn = sn * lax.rsqrt(jnp.maximum(sn, 1e-30))
    losses = jnp.maximum(dp - dn + MARGIN, 0.0)
    out_ref[...] = jnp.sum(losses).reshape(1, 1) / B


_tc_finish = pl.pallas_call(
    _tc_finish_body,
    out_shape=jax.ShapeDtypeStruct((1, 1), jnp.float32),
)


def kernel(outputs, anchors, positives, negatives):
    ia = anchors.astype(jnp.int32)
    ip = positives.astype(jnp.int32)
    inn = negatives.astype(jnp.int32)
    pp, pn = _sc_ssd(outputs, ia, ip, inn)
    res = _tc_finish(pp, pn)
    return res[0, 0]


# speedup vs baseline: 2.9854x; 2.9854x over previous
import functools

import jax
import jax.numpy as jnp
from jax import lax
from jax.experimental import pallas as pl
from jax.experimental.pallas import tpu as pltpu
from jax.experimental.pallas import tpu_sc as plsc

MARGIN = 1.0
EPS = 1e-6

D = 512
B = 16384
L = 16
NW = 32
BPW = B // NW
C = 16
NBUF = 2
NCHUNK = BPW // C
DV = D // L
PROWS = BPW // 8

_mesh = plsc.VectorSubcoreMesh(core_axis_name="c", subcore_axis_name="s")


@functools.partial(
    pl.kernel,
    mesh=_mesh,
    out_type=[
        jax.ShapeDtypeStruct((NW * PROWS, 128), jnp.float32),
        jax.ShapeDtypeStruct((NW * PROWS, 128), jnp.float32),
    ],
    scratch_types=[
        pltpu.VMEM((BPW,), jnp.int32),
        pltpu.VMEM((BPW,), jnp.int32),
        pltpu.VMEM((BPW,), jnp.int32),
        *([pltpu.VMEM((C, D), jnp.float32)] * 6),
        pltpu.VMEM((PROWS, 128), jnp.float32),
        pltpu.VMEM((PROWS, 128), jnp.float32),
        *([pltpu.SemaphoreType.DMA] * 2),
    ],
)
def _sc_ssd(table_hbm, ia_hbm, ip_hbm, in_hbm, outp_hbm, outn_hbm,
            ia_v, ip_v, in_v,
            ra0, rp0, rn0, ra1, rp1, rn1,
            ssdp_v, ssdn_v, sem0, sem1):
    wid = lax.axis_index("s") * 2 + lax.axis_index("c")
    pltpu.sync_copy(ia_hbm.at[pl.ds(wid * BPW, BPW)], ia_v)
    pltpu.sync_copy(ip_hbm.at[pl.ds(wid * BPW, BPW)], ip_v)
    pltpu.sync_copy(in_hbm.at[pl.ds(wid * BPW, BPW)], in_v)

    slots = ((ra0, rp0, rn0, sem0), (ra1, rp1, rn1, sem1))

    def issue(g, slot):
        ra, rp, rn, sem = slot
        base = g * C
        pltpu.async_copy(table_hbm.at[ia_v.at[pl.ds(base, C)]], ra, sem)
        pltpu.async_copy(table_hbm.at[ip_v.at[pl.ds(base, C)]], rp, sem)
        pltpu.async_copy(table_hbm.at[in_v.at[pl.ds(base, C)]], rn, sem)

    def wait_slot(slot):
        ra, rp, rn, sem = slot
        dummy = table_hbm.at[pl.ds(0, C)]
        pltpu.make_async_copy(dummy, ra, sem).wait()
        pltpu.make_async_copy(dummy, rp, sem).wait()
        pltpu.make_async_copy(dummy, rn, sem).wait()

    def compute(g, slot):
        ra, rp, rn, _ = slot
        base = g * C

        def trip_body(i, carry2):
            accp = jnp.zeros((L,), jnp.float32)
            accn = jnp.zeros((L,), jnp.float32)
            for j in range(DV):
                a = ra[i, pl.ds(j * L, L)]
                p = rp[i, pl.ds(j * L, L)]
                n = rn[i, pl.ds(j * L, L)]
                t = a + EPS
                dp = t - p
                dn = t - n
                accp = accp + dp * dp
                accn = accn + dn * dn
            t2 = base + i
            row = t2 // 8
            col = (t2 % 8) * L
            ssdp_v[row, pl.ds(col, L)] = accp
            ssdn_v[row, pl.ds(col, L)] = accn
            return carry2

        lax.fori_loop(0, C, trip_body, 0)

    for s in range(NBUF):
        issue(s, slots[s])

    def body4(gg, carry):
        g0 = NBUF * gg
        for s in range(NBUF):
            g = g0 + s
            wait_slot(slots[s])
            compute(g, slots[s])

            @pl.when(g + NBUF < NCHUNK)
            def _():
                issue(g + NBUF, slots[s])

        return carry

    lax.fori_loop(0, NCHUNK // NBUF, body4, 0)

    pltpu.sync_copy(ssdp_v, outp_hbm.at[pl.ds(wid * PROWS, PROWS)])
    pltpu.sync_copy(ssdn_v, outn_hbm.at[pl.ds(wid * PROWS, PROWS)])


def _tc_finish_body(pp_ref, pn_ref, out_ref):
    lanes = lax.broadcasted_iota(jnp.int32, (128, 8), 0) // L
    groups = lax.broadcasted_iota(jnp.int32, (128, 8), 1)
    g_mat = (lanes == groups).astype(jnp.float32)
    sp = jnp.dot(pp_ref[...], g_mat, preferred_element_type=jnp.float32)
    sn = jnp.dot(pn_ref[...], g_mat, preferred_element_type=jnp.float32)
    dp = sp * lax.rsqrt(jnp.maximum(sp, 1e-30))
    dn = sn * lax.rsqrt(jnp.maximum(sn, 1e-30))
    losses = jnp.maximum(dp - dn + MARGIN, 0.0)
    out_ref[...] = jnp.sum(losses).reshape(1, 1) / B


_tc_finish = pl.pallas_call(
    _tc_finish_body,
    out_shape=jax.ShapeDtypeStruct((1, 1), jnp.float32),
)


def kernel(outputs, anchors, positives, negatives):
    ia = anchors.astype(jnp.int32)
    ip = positives.astype(jnp.int32)
    inn = negatives.astype(jnp.int32)
    pp, pn = _sc_ssd(outputs, ia, ip, inn)
    res = _tc_finish(pp, pn)
    return res[0, 0]
